# baseline (speedup 1.0000x reference)
import functools

import jax
import jax.numpy as jnp
from jax import lax
from jax.experimental import pallas as pl
from jax.experimental.pallas import tpu as pltpu
from jax.experimental.pallas import tpu_sc as plsc

NC, NS, L = 2, 16, 16
NW = NC * NS
CH = 128
NBF = 6
NBB = 6
UNROLL = 6


def _round_up(x, m):
    return (x + m - 1) // m * m


def _body(n_chunks, pad1, pad2,
          idx_hbm, map0_hbm, map1_hbm, map2_hbm,
          emb0_hbm, emb1_hbm, emb2_hbm, out_hbm,
          ixs, f0s, f1s, f2s, bs,
          sp_emb1, sp_emb2,
          semI, semM, semR, semW):
    sid = lax.axis_index("s")
    wid = sid * NC + lax.axis_index("c")
    base = pl.multiple_of(wid * (n_chunks * CH), CH)

    d = bs.shape[1]
    for hbm, sp in ((emb1_hbm, sp_emb1), (emb2_hbm, sp_emb2)):
        rows = hbm.shape[0]
        stripe = rows // NS // 8 * 8
        o = pl.multiple_of(sid * stripe, 8)
        pltpu.sync_copy(hbm.at[pl.ds(o, stripe)], bs.at[pl.ds(0, stripe)])
        pltpu.sync_copy(bs.at[pl.ds(0, stripe)], sp.at[pl.ds(o, stripe)])
        tail = rows - NS * stripe
        if tail:
            @pl.when(sid == 0)
            def _(hbm=hbm, sp=sp, tail=tail, to=NS * stripe):
                pltpu.sync_copy(hbm.at[pl.ds(to, tail)],
                                bs.at[pl.ds(256, tail)])
                pltpu.sync_copy(bs.at[pl.ds(256, tail)],
                                sp.at[pl.ds(to, tail)])

    @pl.when(sid == 0)
    def _():
        for p in range(8):
            for i in range(d // L):
                bs[p, pl.ds(i * L, L)] = jnp.zeros((L,), jnp.float32)
        pltpu.sync_copy(bs.at[pl.ds(0, 8)], sp_emb1.at[pl.ds(pad1, 8)])
        pltpu.sync_copy(bs.at[pl.ds(0, 8)], sp_emb2.at[pl.ds(pad2, 8)])
    plsc.subcore_barrier()

    def valid(c):
        return jnp.logical_and(c >= 0, c < n_chunks)

    def idx_copy(c, bf):
        return pltpu.make_async_copy(
            idx_hbm.at[pl.ds(base + pl.multiple_of(c * CH, CH), CH)],
            ixs.at[pl.ds(bf * CH, CH)], semI.at[bf])

    def map_copies(c, bf):
        sl = ixs.at[pl.ds(bf * CH, CH)]
        fsl = pl.ds(bf * CH, CH)
        return (
            pltpu.make_async_copy(map0_hbm.at[sl], f0s.at[fsl], semM.at[bf]),
            pltpu.make_async_copy(map1_hbm.at[sl], f1s.at[fsl], semM.at[bf]),
            pltpu.make_async_copy(map2_hbm.at[sl], f2s.at[fsl], semM.at[bf]),
        )

    def row_copy(c, bf, bb, which):
        bsl = bs.at[pl.ds(bb * CH, CH)]
        fsl = pl.ds(bf * CH, CH)
        src = (emb0_hbm, sp_emb1, sp_emb2)[which]
        fv = (f0s, f1s, f2s)[which]
        return pltpu.make_async_copy(src.at[fv.at[fsl]], bsl, semR.at[bb])

    def out_copy(c, bb):
        return pltpu.make_async_copy(
            bs.at[pl.ds(bb * CH, CH)],
            out_hbm.at[pl.ds(base + pl.multiple_of(c * CH, CH), CH)],
            semW.at[bb])

    def stage_i(c, bf, bb):
        idx_copy(c, bf).start()

    def stage_m(c, bf, bb):
        idx_copy(c, bf).wait()
        for cp in map_copies(c, bf):
            cp.start()

    def stage_r0(c, bf, bb):
        @pl.when(c >= NBB)
        def _():
            out_copy(c - NBB, bb).wait()
        for cp in map_copies(c, bf):
            cp.wait()
        for i in range(CH // L):
            s = pl.ds(bf * CH + i * L, L)
            v1 = f1s[s]
            f1s[s] = jnp.where(v1 > 0, v1, pad1)
            v2 = f2s[s]
            f2s[s] = jnp.where(v2 > 0, v2, pad2)
        row_copy(c, bf, bb, 0).start()

    def stage_r1(c, bf, bb):
        row_copy(c, bf, bb, 0).wait()
        row_copy(c, bf, bb, 1).start(add=True)

    def stage_r2(c, bf, bb):
        row_copy(c, bf, bb, 1).wait()
        row_copy(c, bf, bb, 2).start(add=True)

    def stage_w(c, bf, bb):
        row_copy(c, bf, bb, 2).wait()
        out_copy(c, bb).start()

    stages = (stage_i, stage_m, stage_r0, stage_r1, stage_r2, stage_w)
    n_outer = (n_chunks + len(stages) - 1 + UNROLL - 1) // UNROLL + 1

    def outer(g, carry):
        for u in range(UNROLL):
            jj = g * UNROLL + u
            for lag, stage in enumerate(stages):
                c = jj - lag
                bf = (u - lag) % NBF
                bb = (u - lag) % NBB

                @pl.when(valid(c))
                def _(c=c, bf=bf, bb=bb, stage=stage):
                    stage(c, bf, bb)
        return carry

    lax.fori_loop(0, n_outer, outer, 0)
    for c in range(n_chunks - NBB, n_chunks):
        out_copy(c, c % NBB).wait()


def _relayout_body(in_ref, out_ref):
    for r in range(out_ref.shape[0]):
        out_ref[r] = in_ref[pl.ds(r * out_ref.shape[1], out_ref.shape[1])]


def kernel(indices, map0, map1, map2, emb0, emb1, emb2):
    b, s = indices.shape
    n = b * s
    d = emb0.shape[1]
    idx = indices.reshape(n).astype(jnp.int32)
    pad1 = _round_up(emb1.shape[0], 8)
    pad2 = _round_up(emb2.shape[0], 8)
    n_chunks = n // (NW * CH)

    out = pl.kernel(
        functools.partial(_body, n_chunks, pad1, pad2),
        out_type=jax.ShapeDtypeStruct((n, d), jnp.float32),
        mesh=plsc.VectorSubcoreMesh(
            core_axis_name="c", subcore_axis_name="s",
            num_cores=NC, num_subcores=NS),
        scratch_types=[
            pltpu.VMEM((NBF * CH,), jnp.int32),
            pltpu.VMEM((NBF * CH,), jnp.int32),
            pltpu.VMEM((NBF * CH,), jnp.int32),
            pltpu.VMEM((NBF * CH,), jnp.int32),
            pltpu.VMEM((NBB * CH, d), jnp.float32),
            pltpu.VMEM_SHARED((pad1 + 8, d), jnp.float32),
            pltpu.VMEM_SHARED((pad2 + 8, d), jnp.float32),
            pltpu.SemaphoreType.DMA((NBF,)),
            pltpu.SemaphoreType.DMA((NBF,)),
            pltpu.SemaphoreType.DMA((NBB,)),
            pltpu.SemaphoreType.DMA((NBB,)),
        ],
    )(idx, map0.astype(jnp.int32), map1.astype(jnp.int32),
      map2.astype(jnp.int32), emb0, emb1, emb2)
    rows_per_block = 8
    return pl.pallas_call(
        _relayout_body,
        grid=(b // rows_per_block,),
        in_specs=[pl.BlockSpec((rows_per_block * s, d), lambda i: (i, 0))],
        out_specs=pl.BlockSpec((rows_per_block, s, d), lambda i: (i, 0, 0)),
        out_shape=jax.ShapeDtypeStruct((b, s, d), jnp.float32),
    )(out)


# speedup vs baseline: 1.5288x; 1.5288x over previous
import functools

import jax
import jax.numpy as jnp
from jax import lax
from jax.experimental import pallas as pl
from jax.experimental.pallas import tpu as pltpu
from jax.experimental.pallas import tpu_sc as plsc

NC, NS, L = 2, 16, 16
NW = NC * NS
CH = 128
NBF = 6
NBB = 6
UNROLL = 6


def _round_up(x, m):
    return (x + m - 1) // m * m


def _body(n_chunks, pad1, pad2,
          idx_hbm, map0_hbm, map1_hbm, map2_hbm,
          emb0_hbm, emb1_hbm, emb2_hbm, out_hbm,
          ixs, f0s, f1s, f2s, bs,
          sp_emb1, sp_emb2,
          semI, semM, semR, semW):
    sid = lax.axis_index("s")
    wid = sid * NC + lax.axis_index("c")
    base = pl.multiple_of(wid * (n_chunks * CH), CH)

    d = bs.shape[1]
    for hbm, sp in ((emb1_hbm, sp_emb1), (emb2_hbm, sp_emb2)):
        rows = hbm.shape[0]
        stripe = rows // NS // 8 * 8
        o = pl.multiple_of(sid * stripe, 8)
        pltpu.sync_copy(hbm.at[pl.ds(o, stripe)], bs.at[pl.ds(0, stripe)])
        pltpu.sync_copy(bs.at[pl.ds(0, stripe)], sp.at[pl.ds(o, stripe)])
        tail = rows - NS * stripe
        if tail:
            @pl.when(sid == 0)
            def _(hbm=hbm, sp=sp, tail=tail, to=NS * stripe):
                pltpu.sync_copy(hbm.at[pl.ds(to, tail)],
                                bs.at[pl.ds(256, tail)])
                pltpu.sync_copy(bs.at[pl.ds(256, tail)],
                                sp.at[pl.ds(to, tail)])

    @pl.when(sid == 0)
    def _():
        for p in range(8):
            for i in range(d // L):
                bs[p, pl.ds(i * L, L)] = jnp.zeros((L,), jnp.float32)
        pltpu.sync_copy(bs.at[pl.ds(0, 8)], sp_emb1.at[pl.ds(pad1, 8)])
        pltpu.sync_copy(bs.at[pl.ds(0, 8)], sp_emb2.at[pl.ds(pad2, 8)])
    plsc.subcore_barrier()

    def valid(c):
        return jnp.logical_and(c >= 0, c < n_chunks)

    def idx_copy(c, bf):
        return pltpu.make_async_copy(
            idx_hbm.at[pl.ds(base + pl.multiple_of(c * CH, CH), CH)],
            ixs.at[pl.ds(bf * CH, CH)], semI.at[bf])

    def map_copies(c, bf):
        sl = ixs.at[pl.ds(bf * CH, CH)]
        fsl = pl.ds(bf * CH, CH)
        return (
            pltpu.make_async_copy(map0_hbm.at[sl], f0s.at[fsl], semM.at[bf]),
            pltpu.make_async_copy(map1_hbm.at[sl], f1s.at[fsl], semM.at[bf]),
            pltpu.make_async_copy(map2_hbm.at[sl], f2s.at[fsl], semM.at[bf]),
        )

    def row_copy(c, bf, bb, which):
        bsl = bs.at[pl.ds(bb * CH, CH)]
        fsl = pl.ds(bf * CH, CH)
        src = (emb0_hbm, sp_emb1, sp_emb2)[which]
        fv = (f0s, f1s, f2s)[which]
        return pltpu.make_async_copy(src.at[fv.at[fsl]], bsl, semR.at[bb])

    def out_copy(c, bb):
        return pltpu.make_async_copy(
            bs.at[pl.ds(bb * CH, CH)],
            out_hbm.at[pl.ds(base + pl.multiple_of(c * CH, CH), CH)],
            semW.at[bb])

    def stage_i(c, bf, bb):
        idx_copy(c, bf).start()

    def stage_m(c, bf, bb):
        idx_copy(c, bf).wait()
        for cp in map_copies(c, bf):
            cp.start()

    def stage_r0(c, bf, bb):
        @pl.when(c >= NBB)
        def _():
            out_copy(c - NBB, bb).wait()
        for cp in map_copies(c, bf):
            cp.wait()
        for i in range(CH // L):
            s = pl.ds(bf * CH + i * L, L)
            v1 = f1s[s]
            f1s[s] = jnp.where(v1 > 0, v1, pad1)
            v2 = f2s[s]
            f2s[s] = jnp.where(v2 > 0, v2, pad2)
        row_copy(c, bf, bb, 0).start()

    def stage_r1(c, bf, bb):
        row_copy(c, bf, bb, 0).wait()
        row_copy(c, bf, bb, 1).start(add=True)

    def stage_r2(c, bf, bb):
        row_copy(c, bf, bb, 1).wait()
        row_copy(c, bf, bb, 2).start(add=True)

    def stage_w(c, bf, bb):
        row_copy(c, bf, bb, 2).wait()
        out_copy(c, bb).start()

    stages = (stage_i, stage_m, stage_r0, stage_r1, stage_r2, stage_w)
    n_outer = (n_chunks + len(stages) - 1 + UNROLL - 1) // UNROLL + 1

    def outer(g, carry):
        for u in range(UNROLL):
            jj = g * UNROLL + u
            for lag, stage in enumerate(stages):
                c = jj - lag
                bf = (u - lag) % NBF
                bb = (u - lag) % NBB

                @pl.when(valid(c))
                def _(c=c, bf=bf, bb=bb, stage=stage):
                    stage(c, bf, bb)
        return carry

    lax.fori_loop(0, n_outer, outer, 0)
    for c in range(n_chunks - NBB, n_chunks):
        out_copy(c, c % NBB).wait()


def kernel(indices, map0, map1, map2, emb0, emb1, emb2):
    b, s = indices.shape
    n = b * s
    d = emb0.shape[1]
    idx = indices.reshape(n).astype(jnp.int32)
    pad1 = _round_up(emb1.shape[0], 8)
    pad2 = _round_up(emb2.shape[0], 8)
    n_chunks = n // (NW * CH)

    out = pl.kernel(
        functools.partial(_body, n_chunks, pad1, pad2),
        out_type=jax.ShapeDtypeStruct((n, d), jnp.float32),
        mesh=plsc.VectorSubcoreMesh(
            core_axis_name="c", subcore_axis_name="s",
            num_cores=NC, num_subcores=NS),
        scratch_types=[
            pltpu.VMEM((NBF * CH,), jnp.int32),
            pltpu.VMEM((NBF * CH,), jnp.int32),
            pltpu.VMEM((NBF * CH,), jnp.int32),
            pltpu.VMEM((NBF * CH,), jnp.int32),
            pltpu.VMEM((NBB * CH, d), jnp.float32),
            pltpu.VMEM_SHARED((pad1 + 8, d), jnp.float32),
            pltpu.VMEM_SHARED((pad2 + 8, d), jnp.float32),
            pltpu.SemaphoreType.DMA((NBF,)),
            pltpu.SemaphoreType.DMA((NBF,)),
            pltpu.SemaphoreType.DMA((NBB,)),
            pltpu.SemaphoreType.DMA((NBB,)),
        ],
    )(idx, map0.astype(jnp.int32), map1.astype(jnp.int32),
      map2.astype(jnp.int32), emb0, emb1, emb2)
    return out.reshape(b, s, d)
